# Initial kernel scaffold; baseline (speedup 1.0000x reference)
import jax
import jax.numpy as jnp
from jax.experimental import pallas as pl


def kernel(seq1, seq2, edge_index, edge_weight, idx, test_idx, W_gcn, b_gcn, alpha, W_bil, b_bil):
    raise NotImplementedError("write your pallas kernel here")


# speedup vs baseline: 8.5618x; 8.5618x over previous
import functools

import jax
import jax.numpy as jnp
from jax import lax
from jax.experimental import pallas as pl
from jax.experimental.pallas import tpu as pltpu
from jax.experimental.pallas import tpu_sc as plsc

N = 10000
E = 320000
D = 128
H = 512
S = 2000
T = 500
NC = 2
NS = 16
NPAD = 10240
AIDX = 2560
EK = 80

_GDN = lax.GatherDimensionNumbers(
    offset_dims=(), collapsed_slice_dims=(0,), start_index_map=(0,))


def _bcast_lane(vec, j):
    return lax.gather(vec, jnp.full((16, 1), j, jnp.int32), _GDN, (1,),
                      mode=lax.GatherScatterMode.PROMISE_IN_BOUNDS)


def _sc_body(xs, srcr, dstr, wr, aidxr, rows_out, deg_out,
             src_v, dst_v, w_v, rows_v, agg_sh, deg_sh, sem):
    core = lax.axis_index("c")
    sub = lax.axis_index("s")
    _ZERO16 = jnp.zeros((16,), jnp.float32)

    def _zrow(f, carry):
        for t in range(D // 16):
            rows_v[f, pl.ds(t * 16, 16)] = _ZERO16
        return carry
    lax.fori_loop(0, EK, _zrow, 0)
    for t in range(EK // 16):
        w_v[pl.ds(t * 16, 16)] = _ZERO16
    zbase = sub * (NPAD // NS)
    for r in range(NPAD // NS // EK):
        pltpu.sync_copy(rows_v, agg_sh.at[pl.ds(zbase + r * EK, EK)])
        pltpu.sync_copy(w_v, deg_sh.at[pl.ds(zbase + r * EK, EK)])
    plsc.subcore_barrier()

    ebase = sub * (E // NS)
    coreN = jnp.full((16,), core * N, jnp.int32)

    def _edge(i, carry):
        b = ebase + i * EK
        pltpu.sync_copy(srcr.at[pl.ds(b, EK)], src_v)
        pltpu.sync_copy(dstr.at[pl.ds(b, EK)], dst_v)
        pltpu.sync_copy(wr.at[pl.ds(b, EK)], w_v)
        for g in range(EK // 16):
            src_v[pl.ds(g * 16, 16)] = src_v[pl.ds(g * 16, 16)] + coreN
        pltpu.async_copy(xs.at[src_v], rows_v, sem).wait()
        for g in range(EK // 16):
            wv = w_v[pl.ds(g * 16, 16)]
            for j in range(16):
                e = g * 16 + j
                bw = _bcast_lane(wv, j)
                for t in range(D // 16):
                    rows_v[e, pl.ds(t * 16, 16)] = (
                        rows_v[e, pl.ds(t * 16, 16)] * bw)
        pltpu.sync_copy(rows_v, agg_sh.at[dst_v], add=True)
        pltpu.sync_copy(w_v, deg_sh.at[dst_v], add=True)
        return carry
    lax.fori_loop(0, (E // NS) // EK, _edge, 0)
    plsc.subcore_barrier()

    obase = sub * (AIDX // NS)
    for q in range(AIDX // NS // EK):
        ob = obase + q * EK
        oo = core * AIDX + ob
        pltpu.sync_copy(aidxr.at[pl.ds(ob, EK)], src_v)
        pltpu.async_copy(agg_sh.at[src_v], rows_v, sem).wait()
        pltpu.sync_copy(rows_v, rows_out.at[pl.ds(oo, EK)])
        pltpu.async_copy(deg_sh.at[src_v], w_v, sem).wait()
        pltpu.sync_copy(w_v, deg_out.at[pl.ds(oo, EK)])


_sc_agg = functools.partial(
    pl.kernel,
    out_type=[
        jax.ShapeDtypeStruct((NC * AIDX, D), jnp.float32),
        jax.ShapeDtypeStruct((NC * AIDX,), jnp.float32),
    ],
    mesh=plsc.VectorSubcoreMesh(
        core_axis_name="c", subcore_axis_name="s",
        num_cores=NC, num_subcores=NS),
    scratch_types=[
        pltpu.VMEM((EK,), jnp.int32),
        pltpu.VMEM((EK,), jnp.int32),
        pltpu.VMEM((EK,), jnp.float32),
        pltpu.VMEM((EK, D), jnp.float32),
        pltpu.VMEM_SHARED((NPAD, D), jnp.float32),
        pltpu.VMEM_SHARED((NPAD,), jnp.float32),
        pltpu.SemaphoreType.DMA,
    ],
)(_sc_body)


def _tc_body(rows1, rows2, deg1, deg2, Wr, br, ar, W0Tr, W1Tr, bbr,
             ret_ref, ret1_ref):
    a = ar[...]
    W = Wr[...]
    b = br[...]
    h1 = jnp.dot(rows1[...], W, preferred_element_type=jnp.float32)
    h1 = h1 + deg1[...] * b
    h1 = jnp.where(h1 >= 0, h1, h1 * a)
    h2 = jnp.dot(rows2[...], W, preferred_element_type=jnp.float32)
    h2 = h2 + deg2[...] * b
    h2 = jnp.where(h2 >= 0, h2, h2 * a)

    c_row = jax.nn.sigmoid(jnp.mean(h1[:S], axis=0, keepdims=True))
    d_row = jax.nn.sigmoid(jnp.mean(h1[S:S + T], axis=0, keepdims=True))

    W0T = W0Tr[...]
    W1T = W1Tr[...]
    tc0 = jnp.dot(c_row, W0T, preferred_element_type=jnp.float32)
    tc1 = jnp.dot(c_row, W1T, preferred_element_type=jnp.float32)
    td0 = jnp.dot(d_row, W0T, preferred_element_type=jnp.float32)
    td1 = jnp.dot(d_row, W1T, preferred_element_type=jnp.float32)

    bb = bbr[...]
    b0 = bb[0:1, 0:1]
    b1 = bb[0:1, 1:2]
    ret_ref[0:S, 0:1] = jnp.sum(h1[:S] * tc0, axis=1, keepdims=True) + b0
    ret_ref[0:S, 1:2] = jnp.sum(h1[:S] * tc1, axis=1, keepdims=True) + b1
    ret_ref[S:2 * S, 0:1] = jnp.sum(h2[:S] * tc0, axis=1, keepdims=True) + b0
    ret_ref[S:2 * S, 1:2] = jnp.sum(h2[:S] * tc1, axis=1, keepdims=True) + b1
    ret1_ref[0:T, 0:1] = (
        jnp.sum(h1[S:S + T] * td0, axis=1, keepdims=True) + b0)
    ret1_ref[0:T, 1:2] = (
        jnp.sum(h1[S:S + T] * td1, axis=1, keepdims=True) + b1)
    ret1_ref[T:2 * T, 0:1] = (
        jnp.sum(h2[S:S + T] * td0, axis=1, keepdims=True) + b0)
    ret1_ref[T:2 * T, 1:2] = (
        jnp.sum(h2[S:S + T] * td1, axis=1, keepdims=True) + b1)


def kernel(seq1, seq2, edge_index, edge_weight, idx, test_idx,
           W_gcn, b_gcn, alpha, W_bil, b_bil):
    src = edge_index[0].astype(jnp.int32)
    dst = edge_index[1].astype(jnp.int32)
    w = edge_weight.astype(jnp.float32)
    xs = jnp.concatenate([seq1, seq2], axis=0)
    aidx = jnp.concatenate([
        idx.astype(jnp.int32), test_idx.astype(jnp.int32),
        jnp.zeros((AIDX - S - T,), jnp.int32)])

    rows_out, deg_out = _sc_agg(xs, src, dst, w, aidx)
    rows1 = rows_out[:AIDX]
    rows2 = rows_out[AIDX:]
    deg1 = deg_out[:AIDX, None]
    deg2 = deg_out[AIDX:, None]

    b2 = b_gcn[None, :]
    a2 = jnp.reshape(alpha.astype(jnp.float32), (1, 1))
    bb2 = b_bil[None, :]
    W0T = W_bil[0].T
    W1T = W_bil[1].T

    ret, ret1 = pl.pallas_call(
        _tc_body,
        out_shape=[
            jax.ShapeDtypeStruct((2 * S, 2), jnp.float32),
            jax.ShapeDtypeStruct((2 * T, 2), jnp.float32),
        ],
    )(rows1, rows2, deg1, deg2, W_gcn, b2, a2, W0T, W1T, bb2)
    return (ret, ret1)
